# baseline (speedup 1.0000x reference)
import functools

import jax
import jax.numpy as jnp
from jax import lax
from jax.experimental import pallas as pl
from jax.experimental.pallas import tpu as pltpu
from jax.experimental.pallas import tpu_sc as plsc

B = 6_400_000
D = 64
V = 552
NC = 2
NS = 16
NW = NC * NS
TI_TOT = B // 128
NTI = 4
TI_MAIN = 1560
CHUNK_I = NTI * 128
NBLK = TI_MAIN // NTI
GROUPS = CHUNK_I // 16
TI_A = NW * TI_MAIN
TI_B = TI_A + 2 * NW


def _emb_body(idx_hbm, tab_hbm, out_hbm, tabv, idxv, tbuf, s_i0, s_i1, s_o0, s_o1):
    s_i = (s_i0, s_i1)
    s_o = (s_o0, s_o1)
    wid = lax.axis_index("s") * NC + lax.axis_index("c")
    ti_lo = wid * TI_MAIN

    pltpu.sync_copy(tab_hbm, tabv)

    def blk_ti0(n):
        return ti_lo + n * NTI

    def start_idx(n, buf):
        n = jnp.minimum(n, NBLK - 1)
        pltpu.async_copy(
            idx_hbm.at[pl.ds(blk_ti0(n) * 128, CHUNK_I)], idxv.at[buf], s_i[buf])

    def wait_idx(buf):
        pltpu.make_async_copy(
            idx_hbm.at[pl.ds(0, CHUNK_I)], idxv.at[buf], s_i[buf]).wait()

    def compute(buf):
        @plsc.parallel_loop(0, GROUPS)
        def grp(j):
            ivec = idxv[buf, pl.ds(j * 16, 16)]
            ti_l = j // 8
            il0 = (j % 8) * 16
            for d0 in range(0, D, 2):
                vals = [plsc.load_gather(tabv.at[d0 + t], [ivec])
                        for t in range(2)]
                for t in range(2):
                    d = d0 + t
                    tbuf[buf, d // 8, ti_l, d % 8, pl.ds(il0, 16)] = vals[t]

    def start_outs(n, buf):
        for td in range(8):
            pltpu.async_copy(
                tbuf.at[buf, td], out_hbm.at[td, pl.ds(blk_ti0(n), NTI)], s_o[buf])

    def wait_outs(buf):
        for td in range(8):
            pltpu.make_async_copy(
                tbuf.at[buf, td], out_hbm.at[td, pl.ds(0, NTI)], s_o[buf]).wait()

    start_idx(0, 0)
    start_idx(1, 1)
    for buf in range(2):
        wait_idx(buf)
        compute(buf)
        start_outs(buf, buf)
        start_idx(buf + 2, buf)

    def body(k, carry):
        for buf in range(2):
            n = 2 * k + buf
            wait_idx(buf)
            compute_n = n
            wait_outs(buf)
            compute(buf)
            start_outs(compute_n, buf)
            start_idx(n + 2, buf)
        return carry

    lax.fori_loop(1, NBLK // 2, body, 0)

    wait_outs(0)
    wait_outs(1)
    wait_idx(0)
    wait_idx(1)

    def tail_block(ti0, nti):
        pltpu.sync_copy(idx_hbm.at[pl.ds(ti0 * 128, nti * 128)],
                        idxv.at[0, pl.ds(0, nti * 128)])

        def grp(j, carry):
            ivec = idxv[0, pl.ds(j * 16, 16)]
            ti_l = j // 8
            il0 = (j % 8) * 16
            for d0 in range(0, D, 2):
                vals = [plsc.load_gather(tabv.at[d0 + t], [ivec])
                        for t in range(2)]
                for t in range(2):
                    d = d0 + t
                    tbuf[0, d // 8, ti_l, d % 8, pl.ds(il0, 16)] = vals[t]
            return carry
        lax.fori_loop(0, 8 * nti, grp, 0)
        for td in range(8):
            pltpu.sync_copy(tbuf.at[0, td, pl.ds(0, nti)],
                            out_hbm.at[td, pl.ds(ti0, nti)])

    tail_block(TI_A + 2 * wid, 2)

    @pl.when(wid < 16)
    def _():
        tail_block(TI_B + wid, 1)


_mesh = plsc.VectorSubcoreMesh(core_axis_name="c", subcore_axis_name="s")

_emb = functools.partial(
    pl.kernel,
    mesh=_mesh,
    out_type=jax.ShapeDtypeStruct((8, TI_TOT, 8, 128), jnp.float32),
    compiler_params=pltpu.CompilerParams(use_tc_tiling_on_sc=False,
                                         needs_layout_passes=False),
    scratch_types=[
        pltpu.VMEM((D, V), jnp.float32),
        pltpu.VMEM((2, CHUNK_I), jnp.int32),
        pltpu.VMEM((2, 8, NTI, 8, 128), jnp.float32),
    ] + [pltpu.SemaphoreType.DMA] * 4,
)(_emb_body)


def kernel(edge_type_indices, table):
    idx = edge_type_indices.astype(jnp.int32)
    tab_t = table.T
    r = _emb(idx, tab_t)
    return r.transpose(0, 2, 1, 3).reshape(D, B).T


# speedup vs baseline: 1.0144x; 1.0144x over previous
import functools

import jax
import jax.numpy as jnp
from jax import lax
from jax.experimental import pallas as pl
from jax.experimental.pallas import tpu as pltpu
from jax.experimental.pallas import tpu_sc as plsc

B = 6_400_000
D = 64
V = 552
NC = 2
NS = 16
NW = NC * NS
TI_TOT = B // 128
NTI = 4
TI_MAIN = 1560
CHUNK_I = NTI * 128
NBLK = TI_MAIN // NTI
GROUPS = CHUNK_I // 16
TI_A = NW * TI_MAIN
TI_B = TI_A + 2 * NW


def _emb_body(idx_hbm, tab_hbm, out_hbm, tabv, idxv, tbuf, s_i0, s_i1, s_o0, s_o1):
    s_i = (s_i0, s_i1)
    s_o = (s_o0, s_o1)
    wid = lax.axis_index("s") * NC + lax.axis_index("c")
    ti_lo = wid * TI_MAIN

    pltpu.sync_copy(tab_hbm, tabv)

    def blk_ti0(n):
        return ti_lo + n * NTI

    def start_idx(n, buf):
        n = jnp.minimum(n, NBLK - 1)
        pltpu.async_copy(
            idx_hbm.at[pl.ds(blk_ti0(n) * 128, CHUNK_I)], idxv.at[buf], s_i[buf])

    def wait_idx(buf):
        pltpu.make_async_copy(
            idx_hbm.at[pl.ds(0, CHUNK_I)], idxv.at[buf], s_i[buf]).wait()

    def compute(buf):
        @plsc.parallel_loop(0, GROUPS)
        def grp(j):
            ivec = idxv[buf, pl.ds(j * 16, 16)]
            ti_l = j // 8
            il0 = (j % 8) * 16
            for d0 in range(0, D, 4):
                vals = [plsc.load_gather(tabv.at[d0 + t], [ivec])
                        for t in range(4)]
                for t in range(4):
                    d = d0 + t
                    tbuf[buf, d // 8, ti_l, d % 8, pl.ds(il0, 16)] = vals[t]

    def start_outs(n, buf):
        for td in range(8):
            pltpu.async_copy(
                tbuf.at[buf, td], out_hbm.at[td, pl.ds(blk_ti0(n), NTI)], s_o[buf])

    def wait_outs(buf):
        for td in range(8):
            pltpu.make_async_copy(
                tbuf.at[buf, td], out_hbm.at[td, pl.ds(0, NTI)], s_o[buf]).wait()

    start_idx(0, 0)
    start_idx(1, 1)
    for buf in range(2):
        wait_idx(buf)
        compute(buf)
        start_outs(buf, buf)
        start_idx(buf + 2, buf)

    def body(k, carry):
        for buf in range(2):
            n = 2 * k + buf
            wait_idx(buf)
            compute_n = n
            wait_outs(buf)
            compute(buf)
            start_outs(compute_n, buf)
            start_idx(n + 2, buf)
        return carry

    lax.fori_loop(1, NBLK // 2, body, 0)

    wait_outs(0)
    wait_outs(1)
    wait_idx(0)
    wait_idx(1)

    def tail_block(ti0, nti):
        pltpu.sync_copy(idx_hbm.at[pl.ds(ti0 * 128, nti * 128)],
                        idxv.at[0, pl.ds(0, nti * 128)])

        def grp(j, carry):
            ivec = idxv[0, pl.ds(j * 16, 16)]
            ti_l = j // 8
            il0 = (j % 8) * 16
            for d0 in range(0, D, 4):
                vals = [plsc.load_gather(tabv.at[d0 + t], [ivec])
                        for t in range(4)]
                for t in range(4):
                    d = d0 + t
                    tbuf[0, d // 8, ti_l, d % 8, pl.ds(il0, 16)] = vals[t]
            return carry
        lax.fori_loop(0, 8 * nti, grp, 0)
        for td in range(8):
            pltpu.sync_copy(tbuf.at[0, td, pl.ds(0, nti)],
                            out_hbm.at[td, pl.ds(ti0, nti)])

    tail_block(TI_A + 2 * wid, 2)

    @pl.when(wid < 16)
    def _():
        tail_block(TI_B + wid, 1)


_mesh = plsc.VectorSubcoreMesh(core_axis_name="c", subcore_axis_name="s")

_emb = functools.partial(
    pl.kernel,
    mesh=_mesh,
    out_type=jax.ShapeDtypeStruct((8, TI_TOT, 8, 128), jnp.float32),
    compiler_params=pltpu.CompilerParams(use_tc_tiling_on_sc=False,
                                         needs_layout_passes=False),
    scratch_types=[
        pltpu.VMEM((D, V), jnp.float32),
        pltpu.VMEM((2, CHUNK_I), jnp.int32),
        pltpu.VMEM((2, 8, NTI, 8, 128), jnp.float32),
    ] + [pltpu.SemaphoreType.DMA] * 4,
)(_emb_body)


def kernel(edge_type_indices, table):
    idx = edge_type_indices.astype(jnp.int32)
    tab_t = table.T
    r = _emb(idx, tab_t)
    return r.transpose(0, 2, 1, 3).reshape(D, B).T
